# baseline (speedup 1.0000x reference)
import functools

import jax
import jax.numpy as jnp
from jax import lax
from jax.experimental import pallas as pl
from jax.experimental.pallas import tpu as pltpu
from jax.experimental.pallas import tpu_sc as plsc

N = 10000
E = 320000
F = 128
H = 128
C = 10
G = 64
EPS = 1e-5

NC = 2
NS = 16
NW = NC * NS
CH = 128
NP = 10240
STRIPE = NP // NS
NSLAB = 8
SS = 10
RPW = NSLAB * SS
EP = NW * RPW * CH

_DOT = functools.partial(
    lax.dot_general,
    preferred_element_type=jnp.float32,
)


def _mm(a, b):
    return _DOT(a, b, dimension_numbers=(((1,), (0,)), ((), ())))


@functools.cache
def _make_seg_sum_kernel():
    mesh = plsc.VectorSubcoreMesh(core_axis_name="c", subcore_axis_name="s")

    @functools.partial(
        pl.kernel,
        out_type=jax.ShapeDtypeStruct((NC, NP, H), jnp.float32),
        mesh=mesh,
        scratch_types=[
            pltpu.VMEM((SS, CH), jnp.int32),
            pltpu.VMEM((SS, CH), jnp.int32),
            pltpu.VMEM((CH, H), jnp.float32),
            pltpu.VMEM((CH, H), jnp.float32),
            pltpu.VMEM_SHARED((NP, H), jnp.float32),
            pltpu.SemaphoreType.DMA,
            pltpu.SemaphoreType.DMA,
            pltpu.SemaphoreType.DMA,
            pltpu.SemaphoreType.DMA,
        ],
    )
    def seg_sum(h_hbm, src_hbm, dst_hbm, zero_hbm, out_hbm,
                src_v, dst_v, rows_a, rows_b, acc,
                gsem_a, gsem_b, ssem_a, ssem_b):
        cid = lax.axis_index("c")
        sid = lax.axis_index("s")
        wid = sid * NC + cid
        pltpu.sync_copy(zero_hbm, acc.at[pl.ds(sid * STRIPE, STRIPE)])
        plsc.subcore_barrier()

        def start_g(buf, sem, i):
            pltpu.async_copy(h_hbm.at[src_v.at[i]], buf, sem)

        def wait_g(buf, sem):
            pltpu.make_async_copy(h_hbm.at[src_v.at[0]], buf, sem).wait()

        def start_s(buf, sem, i):
            pltpu.async_copy(buf, acc.at[dst_v.at[i]], sem, add=True)

        def wait_s(buf, sem):
            pltpu.make_async_copy(buf, acc.at[dst_v.at[0]], sem).wait()

        @pl.loop(0, NSLAB)
        def _(s):
            pltpu.sync_copy(src_hbm.at[wid, s], src_v)
            pltpu.sync_copy(dst_hbm.at[wid, s], dst_v)
            start_g(rows_a, gsem_a, 0)

            @pl.loop(0, SS // 2)
            def _(k):
                i = 2 * k
                wait_g(rows_a, gsem_a)
                start_s(rows_a, ssem_a, i)
                start_g(rows_b, gsem_b, i + 1)
                wait_g(rows_b, gsem_b)
                start_s(rows_b, ssem_b, i + 1)
                wait_s(rows_a, ssem_a)

                @pl.when(i + 2 < SS)
                def _():
                    start_g(rows_a, gsem_a, i + 2)

                wait_s(rows_b, ssem_b)

        plsc.subcore_barrier()
        pltpu.sync_copy(acc.at[pl.ds(sid * STRIPE, STRIPE)],
                        out_hbm.at[cid, pl.ds(sid * STRIPE, STRIPE)])

    return seg_sum


def _seg_sum_kernel(h, src2d, dst2d, zero_rows):
    return _make_seg_sum_kernel()(h, src2d, dst2d, zero_rows)


def _bn_apply(x, g, b):
    def body(x_ref, g_ref, b_ref, o_ref):
        xv = x_ref[...]
        m = jnp.mean(xv, axis=0, keepdims=True)
        v = jnp.mean(xv * xv, axis=0, keepdims=True) - m * m
        o_ref[...] = (xv - m) * lax.rsqrt(v + EPS) * g_ref[...] + b_ref[...]

    return pl.pallas_call(
        body, out_shape=jax.ShapeDtypeStruct((N, F), jnp.float32)
    )(x, g.reshape(1, F), b.reshape(1, F))


def _conv_bn(parts, h, wrel, wroot, bias, g2, b2):
    def body(p_ref, h_ref, wr_ref, wt_ref, b_ref, g_ref, bb_ref, o_ref):
        agg = p_ref[0, :N, :] + p_ref[1, :N, :]
        z = _mm(agg, wr_ref[...]) + _mm(h_ref[...], wt_ref[...]) + b_ref[...]
        z = jnp.maximum(z, 0.0)
        m = jnp.mean(z, axis=0, keepdims=True)
        v = jnp.mean(z * z, axis=0, keepdims=True) - m * m
        o_ref[...] = (z - m) * lax.rsqrt(v + EPS) * g_ref[...] + bb_ref[...]

    return pl.pallas_call(
        body, out_shape=jax.ShapeDtypeStruct((N, H), jnp.float32)
    )(parts, h, wrel, wroot, bias.reshape(1, H),
      g2.reshape(1, H), b2.reshape(1, H))


def _conv_pool_head(parts, h, wrel, wroot, bias, batch_row, linw, linb):
    def body(p_ref, h_ref, wr_ref, wt_ref, b_ref, bt_ref, lw_ref, lb_ref,
             o_ref):
        agg = p_ref[0, :N, :] + p_ref[1, :N, :]
        z = _mm(agg, wr_ref[...]) + _mm(h_ref[...], wt_ref[...]) + b_ref[...]
        oh = (lax.broadcasted_iota(jnp.int32, (G, N), 0)
              == bt_ref[...]).astype(jnp.float32)
        sums = _mm(oh, z)
        counts = jnp.sum(oh, axis=1, keepdims=True)
        pooled = sums / jnp.maximum(counts, 1.0)
        o_ref[...] = _mm(pooled, lw_ref[...]) + lb_ref[...]

    return pl.pallas_call(
        body, out_shape=jax.ShapeDtypeStruct((G, C), jnp.float32)
    )(parts, h, wrel, wroot, bias.reshape(1, H), batch_row,
      linw, linb.reshape(1, C))


def kernel(x, edge_index, batch, params):
    p = params
    pad = EP - E
    pad_idx = jnp.arange(pad, dtype=jnp.int32)
    src2d = jnp.concatenate(
        [edge_index[0], (pad_idx * 131) % N]).reshape(NW, NSLAB, SS, CH)
    dst2d = jnp.concatenate(
        [edge_index[1], N + pad_idx % (NP - N)]).reshape(NW, NSLAB, SS, CH)
    zero_rows = jnp.zeros((STRIPE, H), jnp.float32)
    batch_row = batch.reshape(1, N)

    h = _bn_apply(x, p["bn1_g"], p["bn1_b"])
    for i in (1, 2, 3):
        parts = _seg_sum_kernel(h, src2d, dst2d, zero_rows)
        h = _conv_bn(parts, h, p[f"conv{i}_Wrel"], p[f"conv{i}_Wroot"],
                     p[f"conv{i}_b"], p[f"bn{i+1}_g"], p[f"bn{i+1}_b"])
    parts = _seg_sum_kernel(h, src2d, dst2d, zero_rows)
    return _conv_pool_head(parts, h, p["conv4_Wrel"], p["conv4_Wroot"],
                           p["conv4_b"], batch_row, p["lin_W"], p["lin_b"])


# speedup vs baseline: 1.0903x; 1.0903x over previous
import functools

import jax
import jax.numpy as jnp
from jax import lax
from jax.experimental import pallas as pl
from jax.experimental.pallas import tpu as pltpu
from jax.experimental.pallas import tpu_sc as plsc

N = 10000
E = 320000
F = 128
H = 128
C = 10
G = 64
EPS = 1e-5

NC = 2
NS = 16
NW = NC * NS
CH = 128
NP = 10240
STRIPE = NP // NS
NSLAB = 8
SS = 10
RPW = NSLAB * SS
EP = NW * RPW * CH

_DOT = functools.partial(
    lax.dot_general,
    preferred_element_type=jnp.float32,
)


def _mm(a, b):
    return _DOT(a, b, dimension_numbers=(((1,), (0,)), ((), ())))


@functools.cache
def _make_seg_sum_kernel():
    mesh = plsc.VectorSubcoreMesh(core_axis_name="c", subcore_axis_name="s")

    @functools.partial(
        pl.kernel,
        out_type=jax.ShapeDtypeStruct((NC, NP, H), jnp.float32),
        mesh=mesh,
        scratch_types=[
            pltpu.VMEM((SS, CH), jnp.int32),
            pltpu.VMEM((SS, CH), jnp.int32),
            pltpu.VMEM((CH, H), jnp.float32),
            pltpu.VMEM((CH, H), jnp.float32),
            pltpu.VMEM_SHARED((NP, H), jnp.float32),
            pltpu.SemaphoreType.DMA,
            pltpu.SemaphoreType.DMA,
            pltpu.SemaphoreType.DMA,
            pltpu.SemaphoreType.DMA,
        ],
    )
    def seg_sum(h_hbm, src_hbm, dst_hbm, zero_hbm, out_hbm,
                src_v, dst_v, rows_a, rows_b, acc,
                gsem_a, gsem_b, ssem_a, ssem_b):
        cid = lax.axis_index("c")
        sid = lax.axis_index("s")
        wid = sid * NC + cid
        pltpu.sync_copy(zero_hbm, acc.at[pl.ds(sid * STRIPE, STRIPE)])
        plsc.subcore_barrier()

        def start_g(buf, sem, i):
            pltpu.async_copy(h_hbm.at[src_v.at[i]], buf, sem)

        def wait_g(buf, sem):
            pltpu.make_async_copy(h_hbm.at[src_v.at[0]], buf, sem).wait()

        @pl.loop(0, NSLAB)
        def _(s):
            pltpu.sync_copy(src_hbm.at[wid, s], src_v)
            pltpu.sync_copy(dst_hbm.at[wid, s], dst_v)
            start_g(rows_a, gsem_a, 0)

            @pl.loop(0, SS // 2)
            def _(k):
                i = 2 * k
                start_g(rows_b, gsem_b, i + 1)
                wait_g(rows_a, gsem_a)
                pltpu.sync_copy(rows_a, acc.at[dst_v.at[i]], add=True)

                @pl.when(i + 2 < SS)
                def _():
                    start_g(rows_a, gsem_a, i + 2)

                wait_g(rows_b, gsem_b)
                pltpu.sync_copy(rows_b, acc.at[dst_v.at[i + 1]], add=True)

        plsc.subcore_barrier()
        pltpu.sync_copy(acc.at[pl.ds(sid * STRIPE, STRIPE)],
                        out_hbm.at[cid, pl.ds(sid * STRIPE, STRIPE)])

    return seg_sum


def _seg_sum_kernel(h, src2d, dst2d, zero_rows):
    return _make_seg_sum_kernel()(h, src2d, dst2d, zero_rows)


def _bn_apply(x, g, b):
    def body(x_ref, g_ref, b_ref, o_ref):
        xv = x_ref[...]
        m = jnp.mean(xv, axis=0, keepdims=True)
        v = jnp.mean(xv * xv, axis=0, keepdims=True) - m * m
        o_ref[...] = (xv - m) * lax.rsqrt(v + EPS) * g_ref[...] + b_ref[...]

    return pl.pallas_call(
        body, out_shape=jax.ShapeDtypeStruct((N, F), jnp.float32)
    )(x, g.reshape(1, F), b.reshape(1, F))


def _root_mm(h, wroot, bias):
    def body(h_ref, wt_ref, b_ref, o_ref):
        o_ref[...] = _mm(h_ref[...], wt_ref[...]) + b_ref[...]

    return pl.pallas_call(
        body, out_shape=jax.ShapeDtypeStruct((N, H), jnp.float32)
    )(h, wroot, bias.reshape(1, H))


def _rel_bn(parts, zroot, wrel, g2, b2):
    def body(p_ref, zr_ref, wr_ref, g_ref, bb_ref, o_ref):
        agg = p_ref[0, :N, :] + p_ref[1, :N, :]
        z = _mm(agg, wr_ref[...]) + zr_ref[...]
        z = jnp.maximum(z, 0.0)
        m = jnp.mean(z, axis=0, keepdims=True)
        v = jnp.mean(z * z, axis=0, keepdims=True) - m * m
        o_ref[...] = (z - m) * lax.rsqrt(v + EPS) * g_ref[...] + bb_ref[...]

    return pl.pallas_call(
        body, out_shape=jax.ShapeDtypeStruct((N, H), jnp.float32)
    )(parts, zroot, wrel, g2.reshape(1, H), b2.reshape(1, H))


def _rel_pool_head(parts, zroot, wrel, batch_row, linw, linb):
    def body(p_ref, zr_ref, wr_ref, bt_ref, lw_ref, lb_ref, o_ref):
        agg = p_ref[0, :N, :] + p_ref[1, :N, :]
        z = _mm(agg, wr_ref[...]) + zr_ref[...]
        oh = (lax.broadcasted_iota(jnp.int32, (G, N), 0)
              == bt_ref[...]).astype(jnp.float32)
        sums = _mm(oh, z)
        counts = jnp.sum(oh, axis=1, keepdims=True)
        pooled = sums / jnp.maximum(counts, 1.0)
        o_ref[...] = _mm(pooled, lw_ref[...]) + lb_ref[...]

    return pl.pallas_call(
        body, out_shape=jax.ShapeDtypeStruct((G, C), jnp.float32)
    )(parts, zroot, wrel, batch_row, linw, linb.reshape(1, C))


def kernel(x, edge_index, batch, params):
    p = params
    pad = EP - E
    pad_idx = jnp.arange(pad, dtype=jnp.int32)
    src2d = jnp.concatenate(
        [edge_index[0], (pad_idx * 131) % N]).reshape(NW, NSLAB, SS, CH)
    dst2d = jnp.concatenate(
        [edge_index[1], N + pad_idx % (NP - N)]).reshape(NW, NSLAB, SS, CH)
    zero_rows = jnp.zeros((STRIPE, H), jnp.float32)
    batch_row = batch.reshape(1, N)

    h = _bn_apply(x, p["bn1_g"], p["bn1_b"])
    for i in (1, 2, 3):
        parts = _seg_sum_kernel(h, src2d, dst2d, zero_rows)
        zroot = _root_mm(h, p[f"conv{i}_Wroot"], p[f"conv{i}_b"])
        h = _rel_bn(parts, zroot, p[f"conv{i}_Wrel"],
                    p[f"bn{i+1}_g"], p[f"bn{i+1}_b"])
    parts = _seg_sum_kernel(h, src2d, dst2d, zero_rows)
    zroot = _root_mm(h, p["conv4_Wroot"], p["conv4_b"])
    return _rel_pool_head(parts, zroot, p["conv4_Wrel"], batch_row,
                          p["lin_W"], p["lin_b"])


# speedup vs baseline: 1.1040x; 1.0126x over previous
import functools

import jax
import jax.numpy as jnp
from jax import lax
from jax.experimental import pallas as pl
from jax.experimental.pallas import tpu as pltpu
from jax.experimental.pallas import tpu_sc as plsc

N = 10000
E = 320000
F = 128
H = 128
C = 10
G = 64
EPS = 1e-5

NC = 2
NS = 16
NW = NC * NS
CH = 128
NP = 10240
STRIPE = NP // NS
NSLAB = 8
SS = 10
RPW = NSLAB * SS
EP = NW * RPW * CH

_DOT = functools.partial(
    lax.dot_general,
    preferred_element_type=jnp.float32,
)


def _mm(a, b):
    return _DOT(a, b, dimension_numbers=(((1,), (0,)), ((), ())))


@functools.cache
def _make_seg_sum_kernel():
    mesh = plsc.VectorSubcoreMesh(core_axis_name="c", subcore_axis_name="s")

    @functools.partial(
        pl.kernel,
        out_type=jax.ShapeDtypeStruct((NC, NP, H), jnp.float32),
        mesh=mesh,
        scratch_types=[
            pltpu.VMEM((SS, CH), jnp.int32),
            pltpu.VMEM((SS, CH), jnp.int32),
            pltpu.VMEM((CH, H), jnp.float32),
            pltpu.VMEM((CH, H), jnp.float32),
            pltpu.VMEM_SHARED((NP, H), jnp.float32),
            pltpu.SemaphoreType.DMA,
            pltpu.SemaphoreType.DMA,
            pltpu.SemaphoreType.DMA,
            pltpu.SemaphoreType.DMA,
        ],
    )
    def seg_sum(h_hbm, src_hbm, dst_hbm, zero_hbm, out_hbm,
                src_v, dst_v, rows_a, rows_b, acc,
                gsem_a, gsem_b, ssem_a, ssem_b):
        cid = lax.axis_index("c")
        sid = lax.axis_index("s")
        wid = sid * NC + cid
        pltpu.sync_copy(zero_hbm, acc.at[pl.ds(sid * STRIPE, STRIPE)])
        plsc.subcore_barrier()

        def start_g(buf, sem, i):
            pltpu.async_copy(h_hbm.at[src_v.at[i]], buf, sem)

        def wait_g(buf, sem):
            pltpu.make_async_copy(h_hbm.at[src_v.at[0]], buf, sem).wait()

        @pl.loop(0, NSLAB)
        def _(s):
            pltpu.sync_copy(src_hbm.at[wid, s], src_v)
            pltpu.sync_copy(dst_hbm.at[wid, s], dst_v)
            start_g(rows_a, gsem_a, 0)

            @pl.loop(0, SS // 2)
            def _(k):
                i = 2 * k
                start_g(rows_b, gsem_b, i + 1)
                wait_g(rows_a, gsem_a)
                pltpu.sync_copy(rows_a, acc.at[dst_v.at[i]], add=True)

                @pl.when(i + 2 < SS)
                def _():
                    start_g(rows_a, gsem_a, i + 2)

                wait_g(rows_b, gsem_b)
                pltpu.sync_copy(rows_b, acc.at[dst_v.at[i + 1]], add=True)

        plsc.subcore_barrier()
        pltpu.sync_copy(acc.at[pl.ds(sid * STRIPE, STRIPE)],
                        out_hbm.at[cid, pl.ds(sid * STRIPE, STRIPE)])

    return seg_sum


def _seg_sum_kernel(h, src2d, dst2d, zero_rows):
    return _make_seg_sum_kernel()(h, src2d, dst2d, zero_rows)


def _bn_apply(x, g, b):
    def body(x_ref, g_ref, b_ref, o_ref):
        xv = x_ref[...]
        m = jnp.mean(xv, axis=0, keepdims=True)
        v = jnp.mean(xv * xv, axis=0, keepdims=True) - m * m
        o_ref[...] = (xv - m) * lax.rsqrt(v + EPS) * g_ref[...] + b_ref[...]

    return pl.pallas_call(
        body, out_shape=jax.ShapeDtypeStruct((N, F), jnp.float32)
    )(x, g.reshape(1, F), b.reshape(1, F))


def _conv_bn(parts, h, wrel, wroot, bias, g2, b2):
    def body(p_ref, h_ref, wr_ref, wt_ref, b_ref, g_ref, bb_ref, o_ref):
        agg = p_ref[0, :N, :] + p_ref[1, :N, :]
        z = _mm(agg, wr_ref[...]) + _mm(h_ref[...], wt_ref[...]) + b_ref[...]
        z = jnp.maximum(z, 0.0)
        m = jnp.mean(z, axis=0, keepdims=True)
        v = jnp.mean(z * z, axis=0, keepdims=True) - m * m
        o_ref[...] = (z - m) * lax.rsqrt(v + EPS) * g_ref[...] + bb_ref[...]

    return pl.pallas_call(
        body, out_shape=jax.ShapeDtypeStruct((N, H), jnp.float32)
    )(parts, h, wrel, wroot, bias.reshape(1, H),
      g2.reshape(1, H), b2.reshape(1, H))


def _conv_pool_head(parts, h, wrel, wroot, bias, batch_row, linw, linb):
    def body(p_ref, h_ref, wr_ref, wt_ref, b_ref, bt_ref, lw_ref, lb_ref,
             o_ref):
        agg = p_ref[0, :N, :] + p_ref[1, :N, :]
        z = _mm(agg, wr_ref[...]) + _mm(h_ref[...], wt_ref[...]) + b_ref[...]
        oh = (lax.broadcasted_iota(jnp.int32, (G, N), 0)
              == bt_ref[...]).astype(jnp.float32)
        sums = _mm(oh, z)
        counts = jnp.sum(oh, axis=1, keepdims=True)
        pooled = sums / jnp.maximum(counts, 1.0)
        o_ref[...] = _mm(pooled, lw_ref[...]) + lb_ref[...]

    return pl.pallas_call(
        body, out_shape=jax.ShapeDtypeStruct((G, C), jnp.float32)
    )(parts, h, wrel, wroot, bias.reshape(1, H), batch_row,
      linw, linb.reshape(1, C))


def kernel(x, edge_index, batch, params):
    p = params
    pad = EP - E
    pad_idx = jnp.arange(pad, dtype=jnp.int32)
    src2d = jnp.concatenate(
        [edge_index[0], (pad_idx * 131) % N]).reshape(NW, NSLAB, SS, CH)
    dst2d = jnp.concatenate(
        [edge_index[1], N + pad_idx % (NP - N)]).reshape(NW, NSLAB, SS, CH)
    zero_rows = jnp.zeros((STRIPE, H), jnp.float32)
    batch_row = batch.reshape(1, N)

    h = _bn_apply(x, p["bn1_g"], p["bn1_b"])
    for i in (1, 2, 3):
        parts = _seg_sum_kernel(h, src2d, dst2d, zero_rows)
        h = _conv_bn(parts, h, p[f"conv{i}_Wrel"], p[f"conv{i}_Wroot"],
                     p[f"conv{i}_b"], p[f"bn{i+1}_g"], p[f"bn{i+1}_b"])
    parts = _seg_sum_kernel(h, src2d, dst2d, zero_rows)
    return _conv_pool_head(parts, h, p["conv4_Wrel"], p["conv4_Wroot"],
                           p["conv4_b"], batch_row, p["lin_W"], p["lin_b"])
